# Initial kernel scaffold; baseline (speedup 1.0000x reference)
import jax
import jax.numpy as jnp
from jax.experimental import pallas as pl


def kernel(inds):
    raise NotImplementedError("write your pallas kernel here")


# speedup vs baseline: 87.1486x; 87.1486x over previous
import functools

import jax
import jax.numpy as jnp
from jax import lax
from jax.experimental import pallas as pl
from jax.experimental.pallas import tpu as pltpu
from jax.experimental.pallas import tpu_sc as plsc

AXIS = 1_000_000
PAD = 1_048_576
NC, NS = 2, 16
NW = NC * NS
N_IDX = 16384 * 100
IDX_COLS = 128
IDX_ROWS = N_IDX // IDX_COLS
ROWS_W = IDX_ROWS // NW
SLICE_W = PAD // NS
ZB = 8192
GRP = 16


def _sc_scatter_call(inds2d):
    mesh = plsc.VectorSubcoreMesh(core_axis_name="c", subcore_axis_name="s")

    @functools.partial(
        pl.kernel,
        out_type=jax.ShapeDtypeStruct((NC, NS, SLICE_W), jnp.int32),
        mesh=mesh,
        scratch_types=[
            pltpu.VMEM((ZB,), jnp.int32),
            pltpu.VMEM((IDX_COLS,), jnp.int32),
            pltpu.VMEM((ROWS_W, IDX_COLS), jnp.int32),
            pltpu.VMEM_SHARED((PAD,), jnp.int32),
            pltpu.SemaphoreType.DMA,
        ],
    )
    def k(inds_hbm, out_hbm, zbuf, ones_v, idxbuf, seen_sp, sem):
        c = lax.axis_index("c")
        s = lax.axis_index("s")
        w = c * NS + s

        idx_cp = pltpu.async_copy(inds_hbm.at[pl.ds(w * ROWS_W, ROWS_W)],
                                  idxbuf, sem)

        def zb_body(i, carry):
            zbuf[pl.ds(i * 16, 16)] = jnp.zeros((16,), jnp.int32)
            return carry
        lax.fori_loop(0, ZB // 16, zb_body, 0)
        for j in range(IDX_COLS // 16):
            ones_v[pl.ds(j * 16, 16)] = jnp.ones((16,), jnp.int32)

        for j in range(SLICE_W // ZB):
            pltpu.sync_copy(zbuf,
                            seen_sp.at[pl.ds(s * SLICE_W + j * ZB, ZB)])
        idx_cp.wait()
        plsc.subcore_barrier()

        def group_body(g, carry):
            hs = []
            for j in range(GRP):
                row = idxbuf.at[g * GRP + j]
                hs.append(pltpu.async_copy(ones_v, seen_sp.at[row], sem))
            for h in hs:
                h.wait()
            return carry
        lax.fori_loop(0, ROWS_W // GRP, group_body, 0)
        plsc.subcore_barrier()

        pltpu.sync_copy(seen_sp.at[pl.ds(s * SLICE_W, SLICE_W)],
                        out_hbm.at[c, s])

    return k(inds2d)


def _tc_merge_call(seen2):
    RB = 512
    rows = PAD // 128

    def body(x_ref, seen_ref, cnt_ref):
        i = pl.program_id(0)
        a = x_ref[0] | x_ref[1]
        seen_ref[...] = a != 0

        @pl.when(i == 0)
        def _():
            cnt_ref[...] = jnp.zeros((1, 1), jnp.int32)

        cnt_ref[...] = cnt_ref[...] + jnp.sum(a)

    return pl.pallas_call(
        body,
        grid=(rows // RB,),
        in_specs=[pl.BlockSpec((NC, RB, 128), lambda i: (0, i, 0))],
        out_specs=[
            pl.BlockSpec((RB, 128), lambda i: (i, 0)),
            pl.BlockSpec((1, 1), lambda i: (0, 0)),
        ],
        out_shape=[
            jax.ShapeDtypeStruct((rows, 128), jnp.bool_),
            jax.ShapeDtypeStruct((1, 1), jnp.int32),
        ],
    )(seen2)


def kernel(inds):
    inds2d = inds.reshape(IDX_ROWS, IDX_COLS)
    seen2 = _sc_scatter_call(inds2d)
    seen2 = seen2.reshape(NC, PAD // 128, 128)
    seen_b, cnt = _tc_merge_call(seen2)
    seen = seen_b.reshape(PAD)[:AXIS]
    return seen, cnt[0, 0]


# speedup vs baseline: 94.7360x; 1.0871x over previous
import functools

import jax
import jax.numpy as jnp
from jax import lax
from jax.experimental import pallas as pl
from jax.experimental.pallas import tpu as pltpu
from jax.experimental.pallas import tpu_sc as plsc

AXIS = 1_000_000
PAD = 1_048_576
NC, NS = 2, 16
NW = NC * NS
N_IDX = 16384 * 100
IDX_COLS = 128
IDX_ROWS = N_IDX // IDX_COLS
ROWS_W = IDX_ROWS // NW
SLICE_W = PAD // NS
ZB = 8192
GRP = 16


def _sc_scatter_call(inds2d):
    mesh = plsc.VectorSubcoreMesh(core_axis_name="c", subcore_axis_name="s")

    @functools.partial(
        pl.kernel,
        out_type=jax.ShapeDtypeStruct((NC * PAD,), jnp.int32),
        mesh=mesh,
        scratch_types=[
            pltpu.VMEM((ZB,), jnp.int32),
            pltpu.VMEM((IDX_COLS,), jnp.int32),
            pltpu.VMEM((ROWS_W, IDX_COLS), jnp.int32),
            pltpu.VMEM_SHARED((PAD,), jnp.int32),
            pltpu.SemaphoreType.DMA,
        ],
    )
    def k(inds_hbm, out_hbm, zbuf, ones_v, idxbuf, seen_sp, sem):
        c = lax.axis_index("c")
        s = lax.axis_index("s")
        w = c * NS + s

        idx_cp = pltpu.async_copy(inds_hbm.at[pl.ds(w * ROWS_W, ROWS_W)],
                                  idxbuf, sem)

        def zb_body(i, carry):
            zbuf[pl.ds(i * 16, 16)] = jnp.zeros((16,), jnp.int32)
            return carry
        lax.fori_loop(0, ZB // 16, zb_body, 0)
        for j in range(IDX_COLS // 16):
            ones_v[pl.ds(j * 16, 16)] = jnp.ones((16,), jnp.int32)

        for j in range(SLICE_W // ZB):
            pltpu.sync_copy(zbuf,
                            seen_sp.at[pl.ds(s * SLICE_W + j * ZB, ZB)])
        idx_cp.wait()
        plsc.subcore_barrier()

        def group_body(g, carry):
            hs = []
            for j in range(GRP):
                row = idxbuf.at[g * GRP + j]
                hs.append(pltpu.async_copy(ones_v, seen_sp.at[row], sem))
            for h in hs:
                h.wait()
            return carry
        lax.fori_loop(0, ROWS_W // GRP, group_body, 0)
        plsc.subcore_barrier()

        pltpu.sync_copy(seen_sp.at[pl.ds(s * SLICE_W, SLICE_W)],
                        out_hbm.at[pl.ds(c * PAD + s * SLICE_W, SLICE_W)])

    return k(inds2d)


def _tc_merge_call(seen2f):
    BLK = 65536
    nblk = PAD // BLK

    def body(xa_ref, xb_ref, seen_ref, cnt_ref):
        i = pl.program_id(0)
        a = xa_ref[...] | xb_ref[...]
        seen_ref[...] = a != 0

        @pl.when(i == 0)
        def _():
            cnt_ref[...] = jnp.zeros((1, 1), jnp.int32)

        cnt_ref[...] = cnt_ref[...] + jnp.sum(a)

    return pl.pallas_call(
        body,
        grid=(nblk,),
        in_specs=[
            pl.BlockSpec((BLK,), lambda i: (i,)),
            pl.BlockSpec((BLK,), lambda i: (i + nblk,)),
        ],
        out_specs=[
            pl.BlockSpec((BLK,), lambda i: (i,)),
            pl.BlockSpec((1, 1), lambda i: (0, 0)),
        ],
        out_shape=[
            jax.ShapeDtypeStruct((AXIS,), jnp.bool_),
            jax.ShapeDtypeStruct((1, 1), jnp.int32),
        ],
    )(seen2f, seen2f)


def kernel(inds):
    inds2d = inds.reshape(IDX_ROWS, IDX_COLS)
    seen2f = _sc_scatter_call(inds2d)
    seen_b, cnt = _tc_merge_call(seen2f)
    return seen_b, cnt[0, 0]


# speedup vs baseline: 94.8752x; 1.0015x over previous
import functools

import jax
import jax.numpy as jnp
from jax import lax
from jax.experimental import pallas as pl
from jax.experimental.pallas import tpu as pltpu
from jax.experimental.pallas import tpu_sc as plsc

AXIS = 1_000_000
PAD = 1_048_576
NC, NS = 2, 16
NW = NC * NS
N_IDX = 16384 * 100
IDX_COLS = 128
IDX_ROWS = N_IDX // IDX_COLS
ROWS_W = IDX_ROWS // NW
IDX_W = ROWS_W * IDX_COLS
SLICE_W = PAD // NS
ZB = 8192
GRP = 16


def _sc_scatter_call(inds2d):
    mesh = plsc.VectorSubcoreMesh(core_axis_name="c", subcore_axis_name="s")

    @functools.partial(
        pl.kernel,
        out_type=jax.ShapeDtypeStruct((NC * PAD,), jnp.int32),
        mesh=mesh,
        scratch_types=[
            pltpu.VMEM((ZB,), jnp.int32),
            pltpu.VMEM((IDX_COLS,), jnp.int32),
            pltpu.VMEM((IDX_W,), jnp.int32),
            pltpu.VMEM_SHARED((PAD,), jnp.int32),
            pltpu.SemaphoreType.DMA,
        ],
    )
    def k(inds_hbm, out_hbm, zbuf, ones_v, idxbuf, seen_sp, sem):
        c = lax.axis_index("c")
        s = lax.axis_index("s")
        w = c * NS + s

        idx_cp = pltpu.async_copy(inds_hbm.at[pl.ds(w * IDX_W, IDX_W)],
                                  idxbuf, sem)

        def zb_body(i, carry):
            zbuf[pl.ds(i * 16, 16)] = jnp.zeros((16,), jnp.int32)
            return carry
        lax.fori_loop(0, ZB // 16, zb_body, 0)
        for j in range(IDX_COLS // 16):
            ones_v[pl.ds(j * 16, 16)] = jnp.ones((16,), jnp.int32)

        for j in range(SLICE_W // ZB):
            pltpu.sync_copy(zbuf,
                            seen_sp.at[pl.ds(s * SLICE_W + j * ZB, ZB)])
        idx_cp.wait()
        plsc.subcore_barrier()

        def group_body(g, carry):
            hs = []
            for j in range(GRP):
                row = idxbuf.at[pl.ds((g * GRP + j) * IDX_COLS, IDX_COLS)]
                hs.append(pltpu.async_copy(ones_v, seen_sp.at[row], sem))
            for h in hs:
                h.wait()
            return carry
        lax.fori_loop(0, ROWS_W // GRP, group_body, 0)
        plsc.subcore_barrier()

        pltpu.sync_copy(seen_sp.at[pl.ds(s * SLICE_W, SLICE_W)],
                        out_hbm.at[pl.ds(c * PAD + s * SLICE_W, SLICE_W)])

    return k(inds2d)


def _tc_merge_call(seen2f):
    BLK = 65536
    nblk = PAD // BLK

    def body(xa_ref, xb_ref, seen_ref, cnt_ref):
        i = pl.program_id(0)
        a = xa_ref[...] | xb_ref[...]
        seen_ref[...] = a != 0

        @pl.when(i == 0)
        def _():
            cnt_ref[...] = jnp.zeros((1, 1), jnp.int32)

        cnt_ref[...] = cnt_ref[...] + jnp.sum(a)

    return pl.pallas_call(
        body,
        grid=(nblk,),
        in_specs=[
            pl.BlockSpec((BLK,), lambda i: (i,)),
            pl.BlockSpec((BLK,), lambda i: (i + nblk,)),
        ],
        out_specs=[
            pl.BlockSpec((BLK,), lambda i: (i,)),
            pl.BlockSpec((1, 1), lambda i: (0, 0)),
        ],
        out_shape=[
            jax.ShapeDtypeStruct((AXIS,), jnp.bool_),
            jax.ShapeDtypeStruct((1, 1), jnp.int32),
        ],
    )(seen2f, seen2f)


def kernel(inds):
    seen2f = _sc_scatter_call(inds.reshape(N_IDX))
    seen_b, cnt = _tc_merge_call(seen2f)
    return seen_b, cnt[0, 0]
